# baseline (speedup 1.0000x reference)
import functools

import jax
import jax.numpy as jnp
from jax import lax
from jax.experimental import pallas as pl
from jax.experimental.pallas import tpu as pltpu
from jax.experimental.pallas import tpu_sc as plsc

B = 4096
ED = 768
VDIM = 256
VNUM = 8192
CBN = 4
COMMIT = 0.25
ENC_BB = 1024
DEC_BB = 1024
CH = 512
PATCH_T = 128

_NC = 2
_NS = 16
_NW = _NC * _NS
_BPW = B // _NW


def _enc_body(x_ref, w0, b0, w1, b1, w2, b2, w3, b3, f_ref):
    h = jnp.maximum(jnp.dot(x_ref[:], w0[:]) + b0[:], 0.0)
    h = jnp.maximum(jnp.dot(h, w1[:]) + b1[:], 0.0)
    h = jnp.maximum(jnp.dot(h, w2[:]) + b2[:], 0.0)
    f_ref[:] = jnp.dot(h, w3[:]) + b3[:]


def _dist_body(r_ref, rn_ref, cb_ref, cbn_ref, idx_ref, gap_ref):
    r = r_ref[:]
    mm = lax.dot_general(2.0 * r, cb_ref[:], (((1,), (1,)), ((), ())))
    d = (rn_ref[:] - mm) + cbn_ref[:]
    m = jnp.min(d, axis=1, keepdims=True)
    iota = lax.broadcasted_iota(jnp.int32, (r.shape[0], VNUM), 1)
    idxc = jnp.min(jnp.where(d == m, iota, VNUM), axis=1, keepdims=True)
    idx_ref[:] = idxc
    d2 = jnp.where(iota == idxc, jnp.float32(jnp.inf), d)
    gap_ref[:] = jnp.min(d2, axis=1, keepdims=True) - m


def _dec_body(q_ref, x_ref, w0, b0, w1, b1, w2, b2, w3, b3, acc_ref):
    h = jnp.maximum(jnp.dot(q_ref[:], w0[:]) + b0[:], 0.0)
    h = jnp.maximum(jnp.dot(h, w1[:]) + b1[:], 0.0)
    h = jnp.maximum(jnp.dot(h, w2[:]) + b2[:], 0.0)
    rec = jnp.dot(h, w3[:]) + b3[:]
    diff = rec - x_ref[:]
    s = jnp.sum(diff * diff)
    i = pl.program_id(0)
    prev = jnp.where(i == 0, jnp.zeros((1, 1), jnp.float32), acc_ref[...])
    acc_ref[...] = prev + s


def _make_sc_gather(nrows=B):
    bpw = nrows // _NW

    @functools.partial(
        pl.kernel,
        mesh=plsc.VectorSubcoreMesh(core_axis_name="c", subcore_axis_name="s"),
        out_type=jax.ShapeDtypeStruct((nrows, VDIM), jnp.float32),
        scratch_types=[
            pltpu.VMEM((bpw,), jnp.int32),
            pltpu.VMEM((bpw, VDIM), jnp.float32),
            pltpu.SemaphoreType.DMA,
        ],
    )
    def _sc_gather(table_hbm, idx_hbm, out_hbm, idx_v, rows_v, sem):
        wid = lax.axis_index("s") * _NC + lax.axis_index("c")
        base = wid * bpw
        pltpu.sync_copy(idx_hbm.at[pl.ds(base, bpw)], idx_v)
        pltpu.async_copy(table_hbm.at[idx_v], rows_v, sem).wait()
        pltpu.sync_copy(rows_v, out_hbm.at[pl.ds(base, bpw)])

    return _sc_gather


def _mlp(body, xin, Ws, bs, dims, out_specs, out_shape, bb, extra=None):
    specs = [pl.BlockSpec((bb, dims[0]), lambda i: (i, 0))]
    args = [xin]
    if extra is not None:
        specs.append(pl.BlockSpec((bb, extra.shape[1]), lambda i: (i, 0)))
        args.append(extra)
    for k in range(4):
        specs.append(pl.BlockSpec((dims[k], dims[k + 1]), lambda i: (0, 0)))
        specs.append(pl.BlockSpec((1, dims[k + 1]), lambda i: (0, 0)))
        args.append(Ws[k])
        args.append(bs[k].reshape(1, -1))
    return pl.pallas_call(
        body, grid=(B // bb,),
        in_specs=specs, out_specs=out_specs, out_shape=out_shape,
    )(*args)


def kernel(x, codebooks,
           enc_W0, enc_b0, enc_W1, enc_b1, enc_W2, enc_b2, enc_W3, enc_b3,
           dec_W0, dec_b0, dec_W1, dec_b1, dec_W2, dec_b2, dec_W3, dec_b3):
    f = _mlp(_enc_body, x,
             [enc_W0, enc_W1, enc_W2, enc_W3],
             [enc_b0, enc_b1, enc_b2, enc_b3],
             [ED, 2048, 1024, 512, VDIM],
             pl.BlockSpec((ENC_BB, VDIM), lambda i: (i, 0)),
             jax.ShapeDtypeStruct((B, VDIM), jnp.float32), ENC_BB)

    n = B // CH
    row_spec = pl.BlockSpec((CH, VDIM), lambda i: (i, 0))
    rn_spec = pl.BlockSpec((CH, 1), lambda i: (i, 0))
    cb_spec = pl.BlockSpec((VNUM, VDIM), lambda i: (0, 0))
    cbn_spec = pl.BlockSpec((1, VNUM), lambda i: (0, 0))
    idx_spec = pl.BlockSpec((CH, 1), lambda i: (i, 0))

    gather_h = _make_sc_gather(B // 2)

    def gather(cb, idx_flat):
        return jnp.concatenate(
            [gather_h(cb, idx_flat[:B // 2]), gather_h(cb, idx_flat[B // 2:])],
            axis=0)

    h = jax.nn.relu(x @ enc_W0 + enc_b0)
    h = jax.nn.relu(h @ enc_W1 + enc_b1)
    h = jax.nn.relu(h @ enc_W2 + enc_b2)
    f_x = h @ enc_W3 + enc_b3

    r = f
    idxs = []
    qvs = []
    rnsums = []
    for l in range(CBN):
        cb = codebooks[l]
        cbn = jnp.sum(cb ** 2, axis=1)[None, :]
        rn = jnp.sum(r ** 2, axis=1, keepdims=True)
        idx_l, gap_l = pl.pallas_call(
            _dist_body, grid=(n,),
            in_specs=[row_spec, rn_spec, cb_spec, cbn_spec],
            out_specs=[idx_spec, rn_spec],
            out_shape=[jax.ShapeDtypeStruct((B, 1), jnp.int32),
                       jax.ShapeDtypeStruct((B, 1), jnp.float32)],
        )(r, rn, cb, cbn)
        _, rows = lax.top_k(-gap_l.reshape(B), PATCH_T)
        r_rows = f_x[rows]
        for k in range(l):
            r_rows = r_rows - qvs[k][rows]
        rn_rows = jnp.sum(r_rows ** 2, axis=1, keepdims=True)
        d_rows = (rn_rows - 2.0 * r_rows @ cb.T) + cbn
        idx_rows = jnp.argmin(d_rows, axis=1).astype(jnp.int32)
        idx_flat = idx_l.reshape(B).at[rows].set(idx_rows)
        qv = gather(cb, idx_flat)
        idxs.append(idx_flat.reshape(B, 1))
        qvs.append(qv)
        r = r - qv
        rnsums.append(jnp.sum(r ** 2))

    quantized = qvs[0] + qvs[1] + qvs[2] + qvs[3]
    qst = f + (quantized - f)

    rec = _mlp(_dec_body, qst,
               [dec_W0, dec_W1, dec_W2, dec_W3],
               [dec_b0, dec_b1, dec_b2, dec_b3],
               [VDIM, 512, 1024, 2048, ED],
               pl.BlockSpec((1, 1), lambda i: (0, 0)),
               jax.ShapeDtypeStruct((1, 1), jnp.float32), DEC_BB, extra=x)

    q_loss = (rnsums[0] + rnsums[1] + rnsums[2] + rnsums[3]) \
        * ((1.0 + COMMIT) / (B * VDIM))
    total_loss = q_loss + rec[0, 0] * (1.0 / (B * ED))
    indices = jnp.concatenate(idxs, axis=1)
    return qst, total_loss, indices


# speedup vs baseline: 1.0339x; 1.0339x over previous
import functools

import jax
import jax.numpy as jnp
from jax import lax
from jax.experimental import pallas as pl
from jax.experimental.pallas import tpu as pltpu
from jax.experimental.pallas import tpu_sc as plsc

B = 4096
ED = 768
VDIM = 256
VNUM = 8192
CBN = 4
COMMIT = 0.25
ENC_BB = 1024
DEC_BB = 1024
CH = 512
PATCH_T = 128

_NC = 2
_NS = 16
_NW = _NC * _NS
_BPW = B // _NW


def _enc_body(x_ref, w0, b0, w1, b1, w2, b2, w3, b3, f_ref):
    h = jnp.maximum(jnp.dot(x_ref[:], w0[:]) + b0[:], 0.0)
    h = jnp.maximum(jnp.dot(h, w1[:]) + b1[:], 0.0)
    h = jnp.maximum(jnp.dot(h, w2[:]) + b2[:], 0.0)
    f_ref[:] = jnp.dot(h, w3[:]) + b3[:]


def _dist_body(r_ref, rn_ref, cb_ref, cbn_ref, idx_ref, gap_ref):
    r = r_ref[:]
    mm = lax.dot_general(2.0 * r, cb_ref[:], (((1,), (1,)), ((), ())))
    d = (rn_ref[:] - mm) + cbn_ref[:]
    m = jnp.min(d, axis=1, keepdims=True)
    iota = lax.broadcasted_iota(jnp.int32, (r.shape[0], VNUM), 1)
    idxc = jnp.min(jnp.where(d == m, iota, VNUM), axis=1, keepdims=True)
    idx_ref[:] = idxc
    d2 = jnp.where(iota == idxc, jnp.float32(jnp.inf), d)
    gap_ref[:] = jnp.min(d2, axis=1, keepdims=True) - m


def _dec_body(q_ref, x_ref, w0, b0, w1, b1, w2, b2, w3, b3, acc_ref):
    h = jnp.maximum(jnp.dot(q_ref[:], w0[:]) + b0[:], 0.0)
    h = jnp.maximum(jnp.dot(h, w1[:]) + b1[:], 0.0)
    h = jnp.maximum(jnp.dot(h, w2[:]) + b2[:], 0.0)
    rec = jnp.dot(h, w3[:]) + b3[:]
    diff = rec - x_ref[:]
    s = jnp.sum(diff * diff)
    i = pl.program_id(0)
    prev = jnp.where(i == 0, jnp.zeros((1, 1), jnp.float32), acc_ref[...])
    acc_ref[...] = prev + s


def _make_sc_gather(nrows=B):
    bpw = nrows // _NW

    @functools.partial(
        pl.kernel,
        mesh=plsc.VectorSubcoreMesh(core_axis_name="c", subcore_axis_name="s"),
        out_type=jax.ShapeDtypeStruct((nrows, VDIM), jnp.float32),
        scratch_types=[
            pltpu.VMEM((bpw,), jnp.int32),
            pltpu.VMEM((bpw, VDIM), jnp.float32),
            pltpu.SemaphoreType.DMA,
        ],
    )
    def _sc_gather(table_hbm, idx_hbm, out_hbm, idx_v, rows_v, sem):
        wid = lax.axis_index("s") * _NC + lax.axis_index("c")
        base = wid * bpw
        pltpu.sync_copy(idx_hbm.at[pl.ds(base, bpw)], idx_v)
        pltpu.async_copy(table_hbm.at[idx_v], rows_v, sem).wait()
        pltpu.sync_copy(rows_v, out_hbm.at[pl.ds(base, bpw)])

    return _sc_gather


def _mlp(body, xin, Ws, bs, dims, out_specs, out_shape, bb, extra=None):
    specs = [pl.BlockSpec((bb, dims[0]), lambda i: (i, 0))]
    args = [xin]
    if extra is not None:
        specs.append(pl.BlockSpec((bb, extra.shape[1]), lambda i: (i, 0)))
        args.append(extra)
    for k in range(4):
        specs.append(pl.BlockSpec((dims[k], dims[k + 1]), lambda i: (0, 0)))
        specs.append(pl.BlockSpec((1, dims[k + 1]), lambda i: (0, 0)))
        args.append(Ws[k])
        args.append(bs[k].reshape(1, -1))
    return pl.pallas_call(
        body, grid=(B // bb,),
        in_specs=specs, out_specs=out_specs, out_shape=out_shape,
    )(*args)


def kernel(x, codebooks,
           enc_W0, enc_b0, enc_W1, enc_b1, enc_W2, enc_b2, enc_W3, enc_b3,
           dec_W0, dec_b0, dec_W1, dec_b1, dec_W2, dec_b2, dec_W3, dec_b3):
    f = _mlp(_enc_body, x,
             [enc_W0, enc_W1, enc_W2, enc_W3],
             [enc_b0, enc_b1, enc_b2, enc_b3],
             [ED, 2048, 1024, 512, VDIM],
             pl.BlockSpec((ENC_BB, VDIM), lambda i: (i, 0)),
             jax.ShapeDtypeStruct((B, VDIM), jnp.float32), ENC_BB)

    n = B // CH
    row_spec = pl.BlockSpec((CH, VDIM), lambda i: (i, 0))
    rn_spec = pl.BlockSpec((CH, 1), lambda i: (i, 0))
    cb_spec = pl.BlockSpec((VNUM, VDIM), lambda i: (0, 0))
    cbn_spec = pl.BlockSpec((1, VNUM), lambda i: (0, 0))
    idx_spec = pl.BlockSpec((CH, 1), lambda i: (i, 0))

    gather = _make_sc_gather()

    h = jax.nn.relu(x @ enc_W0 + enc_b0)
    h = jax.nn.relu(h @ enc_W1 + enc_b1)
    h = jax.nn.relu(h @ enc_W2 + enc_b2)
    f_x = h @ enc_W3 + enc_b3

    r = f
    idxs = []
    qvs = []
    rnsums = []
    for l in range(CBN):
        cb = codebooks[l]
        cbn = jnp.sum(cb ** 2, axis=1)[None, :]
        rn = jnp.sum(r ** 2, axis=1, keepdims=True)
        idx_l, gap_l = pl.pallas_call(
            _dist_body, grid=(n,),
            in_specs=[row_spec, rn_spec, cb_spec, cbn_spec],
            out_specs=[idx_spec, rn_spec],
            out_shape=[jax.ShapeDtypeStruct((B, 1), jnp.int32),
                       jax.ShapeDtypeStruct((B, 1), jnp.float32)],
        )(r, rn, cb, cbn)
        _, rows = lax.top_k(-gap_l.reshape(B), PATCH_T)
        r_rows = f_x[rows]
        for k in range(l):
            r_rows = r_rows - qvs[k][rows]
        rn_rows = jnp.sum(r_rows ** 2, axis=1, keepdims=True)
        d_rows = (rn_rows - 2.0 * r_rows @ cb.T) + cbn
        idx_rows = jnp.argmin(d_rows, axis=1).astype(jnp.int32)
        idx_flat = idx_l.reshape(B).at[rows].set(idx_rows)
        qv = gather(cb, idx_flat)
        idxs.append(idx_flat.reshape(B, 1))
        qvs.append(qv)
        r = r - qv
        rnsums.append(jnp.sum(r ** 2))

    quantized = qvs[0] + qvs[1] + qvs[2] + qvs[3]
    qst = f + (quantized - f)

    rec = _mlp(_dec_body, qst,
               [dec_W0, dec_W1, dec_W2, dec_W3],
               [dec_b0, dec_b1, dec_b2, dec_b3],
               [VDIM, 512, 1024, 2048, ED],
               pl.BlockSpec((1, 1), lambda i: (0, 0)),
               jax.ShapeDtypeStruct((1, 1), jnp.float32), DEC_BB, extra=x)

    q_loss = (rnsums[0] + rnsums[1] + rnsums[2] + rnsums[3]) \
        * ((1.0 + COMMIT) / (B * VDIM))
    total_loss = q_loss + rec[0, 0] * (1.0 / (B * ED))
    indices = jnp.concatenate(idxs, axis=1)
    return qst, total_loss, indices
